# Initial kernel scaffold; baseline (speedup 1.0000x reference)
import jax
import jax.numpy as jnp
from jax.experimental import pallas as pl


def kernel(h, segment_ids):
    raise NotImplementedError("write your pallas kernel here")


# speedup vs baseline: 4.7387x; 4.7387x over previous
import functools

import jax
import jax.numpy as jnp
from jax import lax
from jax.experimental import pallas as pl
from jax.experimental.pallas import tpu as pltpu
from jax.experimental.pallas import tpu_sc as plsc

N = 100000
D = 128
S = 1024
NC = 2
NS = 16
NW = NC * NS
BLK = 128
NFULL = N // BLK
TAIL = N - NFULL * BLK
KMAX = (NFULL + NW) // NW
SEG_PER_TILE = S // NS
CW = 128

_mesh = plsc.VectorSubcoreMesh(core_axis_name="c", subcore_axis_name="s")


@functools.partial(
    pl.kernel,
    out_type=(
        jax.ShapeDtypeStruct((NC, S, D), jnp.float32),
        jax.ShapeDtypeStruct((NC, S, CW), jnp.float32),
    ),
    mesh=_mesh,
    scratch_types=[
        pltpu.VMEM((1, BLK), jnp.int32),
        pltpu.VMEM((1, TAIL), jnp.int32),
        pltpu.VMEM((BLK, D), jnp.float32),
        pltpu.VMEM((TAIL, D), jnp.float32),
        pltpu.VMEM((BLK, CW), jnp.float32),
        pltpu.VMEM((SEG_PER_TILE, D), jnp.float32),
        pltpu.VMEM_SHARED((S, D), jnp.float32),
        pltpu.VMEM_SHARED((S, CW), jnp.float32),
    ],
)
def _segsum_sc(h_hbm, ids_hbm, psum_hbm, pcnt_hbm,
               ids_v, idt_v, rows_v, rowt_v, ones_v, zsum_v,
               sums_sh, cnts_sh):
    cid = lax.axis_index("c")
    sid = lax.axis_index("s")
    wid = cid * NS + sid

    zero16 = jnp.zeros((16,), jnp.float32)
    one16 = jnp.ones((16,), jnp.float32)

    @pl.loop(0, SEG_PER_TILE)
    def _(i):
        @pl.loop(0, D, step=16)
        def _(j):
            zsum_v[i, pl.ds(j, 16)] = zero16

    @pl.loop(0, BLK)
    def _(i):
        @pl.loop(0, CW, step=16)
        def _(j):
            ones_v[i, pl.ds(j, 16)] = one16

    pltpu.sync_copy(zsum_v, sums_sh.at[pl.ds(sid * SEG_PER_TILE, SEG_PER_TILE)])
    pltpu.sync_copy(zsum_v, cnts_sh.at[pl.ds(sid * SEG_PER_TILE, SEG_PER_TILE)])

    plsc.subcore_barrier()

    @pl.loop(0, KMAX)
    def _(k):
        b = wid + k * NW

        @pl.when(b < NFULL)
        def _():
            base = b * BLK
            pltpu.sync_copy(ids_hbm.at[pl.ds(base, BLK)], ids_v.at[0])
            pltpu.sync_copy(h_hbm.at[pl.ds(base, BLK)], rows_v)
            pltpu.sync_copy(rows_v, sums_sh.at[ids_v.at[0]], add=True)
            pltpu.sync_copy(ones_v, cnts_sh.at[ids_v.at[0]], add=True)

        @pl.when(b == NFULL)
        def _():
            base = NFULL * BLK
            pltpu.sync_copy(ids_hbm.at[pl.ds(base, TAIL)], idt_v.at[0])
            pltpu.sync_copy(h_hbm.at[pl.ds(base, TAIL)], rowt_v)
            pltpu.sync_copy(rowt_v, sums_sh.at[idt_v.at[0]], add=True)
            pltpu.sync_copy(ones_v.at[pl.ds(0, TAIL)], cnts_sh.at[idt_v.at[0]],
                            add=True)

    plsc.subcore_barrier()

    lo = sid * SEG_PER_TILE
    pltpu.sync_copy(sums_sh.at[pl.ds(lo, SEG_PER_TILE)],
                    psum_hbm.at[cid, pl.ds(lo, SEG_PER_TILE)])
    pltpu.sync_copy(cnts_sh.at[pl.ds(lo, SEG_PER_TILE)],
                    pcnt_hbm.at[cid, pl.ds(lo, SEG_PER_TILE)])


def _combine_tc(psum_ref, pcnt_ref, out_ref):
    sums = psum_ref[0] + psum_ref[1]
    cnts = pcnt_ref[0] + pcnt_ref[1]
    cnt = jnp.maximum(cnts[:, 0:1], 1.0)
    out_ref[...] = sums / cnt


@jax.jit
def kernel(h, segment_ids):
    ids = segment_ids.astype(jnp.int32)
    psum, pcnt = _segsum_sc(h, ids)
    return pl.pallas_call(
        _combine_tc,
        out_shape=jax.ShapeDtypeStruct((S, D), jnp.float32),
    )(psum, pcnt)


# speedup vs baseline: 5.5487x; 1.1709x over previous
import dataclasses
import functools

import jax
import jax.numpy as jnp
from jax import lax
from jax.experimental import pallas as pl
from jax.experimental.pallas import tpu as pltpu
from jax.experimental.pallas import tpu_sc as plsc

N = 100000
D = 128
S = 1024
NC = 2
NS = 16
NW = NC * NS
BLK = 128
NFULL = N // BLK
TAIL = N - NFULL * BLK
KMAX = (NFULL + NW) // NW
SEG_PER_TILE = S // NS
L = 16

_mesh = plsc.VectorSubcoreMesh(core_axis_name="c", subcore_axis_name="s")

_cp = pltpu.CompilerParams()
if "needs_layout_passes" in pltpu.CompilerParams.__dataclass_fields__:
    _cp = dataclasses.replace(_cp, needs_layout_passes=False)


@functools.partial(
    pl.kernel,
    compiler_params=_cp,
    out_type=(
        jax.ShapeDtypeStruct((NC, S, D), jnp.float32),
        jax.ShapeDtypeStruct((NW, S), jnp.float32),
    ),
    mesh=_mesh,
    scratch_types=[
        pltpu.VMEM((1, BLK), jnp.int32),
        pltpu.VMEM((1, TAIL), jnp.int32),
        pltpu.VMEM((BLK, D), jnp.float32),
        pltpu.VMEM((TAIL, D), jnp.float32),
        pltpu.VMEM((S,), jnp.float32),
        pltpu.VMEM((SEG_PER_TILE, D), jnp.float32),
        pltpu.VMEM_SHARED((S, D), jnp.float32),
    ],
)
def _segsum_sc(h_hbm, ids_hbm, psum_hbm, pcnt_hbm,
               ids_v, idt_v, rows_v, rowt_v, hist_v, zsum_v, sums_sh):
    cid = lax.axis_index("c")
    sid = lax.axis_index("s")
    wid = cid * NS + sid

    zero16 = jnp.zeros((L,), jnp.float32)
    one16 = jnp.ones((L,), jnp.float32)

    @pl.loop(0, S, step=L)
    def _(i):
        hist_v[pl.ds(i, L)] = zero16

    @pl.loop(0, SEG_PER_TILE)
    def _(i):
        @pl.loop(0, D, step=L)
        def _(j):
            zsum_v[i, pl.ds(j, L)] = zero16

    pltpu.sync_copy(zsum_v, sums_sh.at[pl.ds(sid * SEG_PER_TILE, SEG_PER_TILE)])

    plsc.subcore_barrier()

    @pl.loop(0, KMAX)
    def _(k):
        b = wid + k * NW

        @pl.when(b < NFULL)
        def _():
            base = b * BLK
            pltpu.sync_copy(ids_hbm.at[pl.ds(base, BLK)], ids_v.at[0])
            pltpu.sync_copy(h_hbm.at[pl.ds(base, BLK)], rows_v)
            pltpu.sync_copy(rows_v, sums_sh.at[ids_v.at[0]], add=True)

            @pl.loop(0, BLK, step=L)
            def _(l):
                idx = ids_v[0, pl.ds(l, L)]
                plsc.addupdate_scatter(hist_v, [idx], one16)

        @pl.when(b == NFULL)
        def _():
            base = NFULL * BLK
            pltpu.sync_copy(ids_hbm.at[pl.ds(base, TAIL)], idt_v.at[0])
            pltpu.sync_copy(h_hbm.at[pl.ds(base, TAIL)], rowt_v)
            pltpu.sync_copy(rowt_v, sums_sh.at[idt_v.at[0]], add=True)

            @pl.loop(0, TAIL, step=L)
            def _(l):
                idx = idt_v[0, pl.ds(l, L)]
                plsc.addupdate_scatter(hist_v, [idx], one16)

    plsc.subcore_barrier()

    lo = sid * SEG_PER_TILE
    pltpu.sync_copy(sums_sh.at[pl.ds(lo, SEG_PER_TILE)],
                    psum_hbm.at[cid, pl.ds(lo, SEG_PER_TILE)])
    pltpu.sync_copy(hist_v, pcnt_hbm.at[wid])


def _combine_tc(psum_ref, pcnt_ref, out_ref):
    sums = psum_ref[0] + psum_ref[1]
    cnt = jnp.maximum(jnp.sum(pcnt_ref[...], axis=0), 1.0)
    out_ref[...] = sums / cnt[:, None]


@jax.jit
def kernel(h, segment_ids):
    ids = segment_ids.astype(jnp.int32)
    psum, pcnt = _segsum_sc(h, ids)
    return pl.pallas_call(
        _combine_tc,
        out_shape=jax.ShapeDtypeStruct((S, D), jnp.float32),
    )(psum, pcnt)


# speedup vs baseline: 8.5349x; 1.5382x over previous
import dataclasses
import functools

import jax
import jax.numpy as jnp
from jax import lax
from jax.experimental import pallas as pl
from jax.experimental.pallas import tpu as pltpu
from jax.experimental.pallas import tpu_sc as plsc

N = 100000
D = 128
S = 1024
NC = 2
NS = 16
NW = NC * NS
BLK = 128
NFULL = N // BLK
TAIL = N - NFULL * BLK
KPW = NFULL // NW
REM = NFULL % NW
KMAX = KPW + 1
SEG_PER_TILE = S // NS
L = 16

_mesh = plsc.VectorSubcoreMesh(core_axis_name="c", subcore_axis_name="s")

_cp = pltpu.CompilerParams()
if "needs_layout_passes" in pltpu.CompilerParams.__dataclass_fields__:
    _cp = dataclasses.replace(_cp, needs_layout_passes=False)


@functools.partial(
    pl.kernel,
    compiler_params=_cp,
    out_type=(
        jax.ShapeDtypeStruct((NC, S, D), jnp.float32),
        jax.ShapeDtypeStruct((NW, S), jnp.float32),
    ),
    mesh=_mesh,
    scratch_types=[
        pltpu.VMEM((KMAX, BLK), jnp.int32),
        pltpu.VMEM((BLK, D), jnp.float32),
        pltpu.VMEM((BLK, D), jnp.float32),
        pltpu.VMEM((1, TAIL), jnp.int32),
        pltpu.VMEM((TAIL, D), jnp.float32),
        pltpu.VMEM((S,), jnp.float32),
        pltpu.VMEM((SEG_PER_TILE, D), jnp.float32),
        pltpu.VMEM_SHARED((S, D), jnp.float32),
        pltpu.SemaphoreType.DMA,
        pltpu.SemaphoreType.DMA,
        pltpu.SemaphoreType.DMA,
    ],
)
def _segsum_sc(h_hbm, idsb_hbm, ids_hbm, psum_hbm, pcnt_hbm,
               ids_v, rows0_v, rows1_v, idt_v, rowt_v, hist_v, zsum_v,
               sums_sh, sem_ids, sr0, sr1):
    cid = lax.axis_index("c")
    sid = lax.axis_index("s")
    wid = cid * NS + sid
    start_w = wid * KPW + jnp.minimum(wid, REM)
    cnt_w = jnp.where(wid < REM, KPW + 1, KPW)

    zero16 = jnp.zeros((L,), jnp.float32)
    one16 = jnp.ones((L,), jnp.float32)

    rows_v = (rows0_v, rows1_v)
    sem_in = (sr0, sr1)

    def _ids_desc():
        return pltpu.make_async_copy(idsb_hbm.at[wid], ids_v, sem_ids)

    def _in_desc(k):
        return pltpu.make_async_copy(
            h_hbm.at[pl.ds((start_w + k) * BLK, BLK)], rows_v[k % 2],
            sem_in[k % 2])

    def start_in(k):
        @pl.when(k < cnt_w)
        def _():
            _in_desc(k).start()

    _ids_desc().start()
    start_in(0)

    @pl.loop(0, S, step=L)
    def _(i):
        hist_v[pl.ds(i, L)] = zero16

    @pl.loop(0, SEG_PER_TILE)
    def _(i):
        @pl.loop(0, D, step=L)
        def _(j):
            zsum_v[i, pl.ds(j, L)] = zero16

    pltpu.sync_copy(zsum_v, sums_sh.at[pl.ds(sid * SEG_PER_TILE, SEG_PER_TILE)])

    plsc.subcore_barrier()

    _ids_desc().wait()

    for k in range(KMAX):
        if k + 1 < KMAX:
            start_in(k + 1)

        @pl.when(k < cnt_w)
        def _(k=k):
            _in_desc(k).wait()
            pltpu.sync_copy(rows_v[k % 2], sums_sh.at[ids_v.at[k]], add=True)

            @pl.loop(0, BLK, step=L)
            def _(l):
                idx = ids_v[k, pl.ds(l, L)]
                plsc.addupdate_scatter(hist_v, [idx], one16)

    @pl.when(wid == NW - 1)
    def _():
        base = NFULL * BLK
        pltpu.sync_copy(ids_hbm.at[pl.ds(base, TAIL)], idt_v.at[0])
        pltpu.sync_copy(h_hbm.at[pl.ds(base, TAIL)], rowt_v)
        pltpu.sync_copy(rowt_v, sums_sh.at[idt_v.at[0]], add=True)

        @pl.loop(0, TAIL, step=L)
        def _(l):
            idx = idt_v[0, pl.ds(l, L)]
            plsc.addupdate_scatter(hist_v, [idx], one16)

    plsc.subcore_barrier()

    lo = sid * SEG_PER_TILE
    pltpu.sync_copy(sums_sh.at[pl.ds(lo, SEG_PER_TILE)],
                    psum_hbm.at[cid, pl.ds(lo, SEG_PER_TILE)])
    pltpu.sync_copy(hist_v, pcnt_hbm.at[wid])


def _combine_tc(psum_ref, pcnt_ref, out_ref):
    sums = psum_ref[0] + psum_ref[1]
    cnt = jnp.maximum(jnp.sum(pcnt_ref[...], axis=0), 1.0)
    out_ref[...] = sums / cnt[:, None]


@jax.jit
def kernel(h, segment_ids):
    ids = segment_ids.astype(jnp.int32)
    ids_pad = jnp.concatenate(
        [ids[:NFULL * BLK], jnp.zeros((NW * KMAX * BLK - NFULL * BLK,),
                                      jnp.int32)]).reshape(NW * KMAX, BLK)
    w = jnp.arange(NW, dtype=jnp.int32)
    starts = w * KPW + jnp.minimum(w, REM)
    blk_idx = starts[:, None] + jnp.arange(KMAX, dtype=jnp.int32)[None, :]
    idsb = ids_pad[blk_idx]
    psum, pcnt = _segsum_sc(h, idsb, ids)
    return pl.pallas_call(
        _combine_tc,
        out_shape=jax.ShapeDtypeStruct((S, D), jnp.float32),
    )(psum, pcnt)


# speedup vs baseline: 8.7362x; 1.0236x over previous
import dataclasses
import functools

import jax
import jax.numpy as jnp
from jax import lax
from jax.experimental import pallas as pl
from jax.experimental.pallas import tpu as pltpu
from jax.experimental.pallas import tpu_sc as plsc

N = 100000
D = 128
S = 1024
NC = 2
NS = 16
NW = NC * NS
BLK = 128
NFULL = N // BLK
TAIL = N - NFULL * BLK
KPW = NFULL // NW
REM = NFULL % NW
KMAX = KPW + 1
SEG_PER_TILE = S // NS
L = 16

_mesh = plsc.VectorSubcoreMesh(core_axis_name="c", subcore_axis_name="s")

_cp = pltpu.CompilerParams()
if "needs_layout_passes" in pltpu.CompilerParams.__dataclass_fields__:
    _cp = dataclasses.replace(_cp, needs_layout_passes=False)


@functools.partial(
    pl.kernel,
    compiler_params=_cp,
    out_type=(
        jax.ShapeDtypeStruct((NC, S, D), jnp.float32),
        jax.ShapeDtypeStruct((NW, S), jnp.float32),
    ),
    mesh=_mesh,
    scratch_types=[
        pltpu.VMEM((1, BLK), jnp.int32),
        pltpu.VMEM((1, BLK), jnp.int32),
        pltpu.VMEM((BLK, D), jnp.float32),
        pltpu.VMEM((BLK, D), jnp.float32),
        pltpu.VMEM((1, TAIL), jnp.int32),
        pltpu.VMEM((TAIL, D), jnp.float32),
        pltpu.VMEM((S,), jnp.float32),
        pltpu.VMEM((SEG_PER_TILE, D), jnp.float32),
        pltpu.VMEM_SHARED((S, D), jnp.float32),
        pltpu.SemaphoreType.DMA,
        pltpu.SemaphoreType.DMA,
        pltpu.SemaphoreType.DMA,
        pltpu.SemaphoreType.DMA,
        pltpu.SemaphoreType.DMA,
        pltpu.SemaphoreType.DMA,
    ],
)
def _segsum_sc(h_hbm, ids_hbm, psum_hbm, pcnt_hbm,
               id0_v, id1_v, rows0_v, rows1_v, idt_v, rowt_v, hist_v, zsum_v,
               sums_sh, si0, si1, sr0, sr1, ss0, ss1):
    cid = lax.axis_index("c")
    sid = lax.axis_index("s")
    wid = cid * NS + sid
    start_w = wid * KPW + jnp.minimum(wid, REM)
    cnt_w = jnp.where(wid < REM, KPW + 1, KPW)

    zero16 = jnp.zeros((L,), jnp.float32)
    one16 = jnp.ones((L,), jnp.float32)

    ids_v = (id0_v, id1_v)
    rows_v = (rows0_v, rows1_v)
    sem_id = (si0, si1)
    sem_in = (sr0, sr1)
    sem_sc = (ss0, ss1)

    def _id_desc(k):
        return pltpu.make_async_copy(
            ids_hbm.at[pl.ds((start_w + k) * BLK, BLK)], ids_v[k % 2].at[0],
            sem_id[k % 2])

    def _in_desc(k):
        return pltpu.make_async_copy(
            h_hbm.at[pl.ds((start_w + k) * BLK, BLK)], rows_v[k % 2],
            sem_in[k % 2])

    def _sc_desc(k):
        return pltpu.make_async_copy(
            rows_v[k % 2], sums_sh.at[ids_v[k % 2].at[0]], sem_sc[k % 2])

    def start_in(k):
        @pl.when(k < cnt_w)
        def _():
            _id_desc(k).start()
            _in_desc(k).start()

    def wait_sc(k):
        @pl.when(k < cnt_w)
        def _():
            _sc_desc(k).wait()

    start_in(0)

    @pl.loop(0, S, step=L)
    def _(i):
        hist_v[pl.ds(i, L)] = zero16

    @pl.loop(0, SEG_PER_TILE)
    def _(i):
        @pl.loop(0, D, step=L)
        def _(j):
            zsum_v[i, pl.ds(j, L)] = zero16

    pltpu.sync_copy(zsum_v, sums_sh.at[pl.ds(sid * SEG_PER_TILE, SEG_PER_TILE)])

    plsc.subcore_barrier()

    for k in range(KMAX):
        if k + 1 < KMAX:
            if k >= 1:
                wait_sc(k - 1)
            start_in(k + 1)

        @pl.when(k < cnt_w)
        def _(k=k):
            _id_desc(k).wait()
            _in_desc(k).wait()
            pltpu.async_copy(rows_v[k % 2], sums_sh.at[ids_v[k % 2].at[0]],
                             sem_sc[k % 2], add=True)

            @pl.loop(0, BLK, step=L)
            def _(l):
                idx = ids_v[k % 2][0, pl.ds(l, L)]
                plsc.addupdate_scatter(hist_v, [idx], one16)
    wait_sc(KMAX - 2)
    wait_sc(KMAX - 1)

    @pl.when(wid == NW - 1)
    def _():
        base = NFULL * BLK
        pltpu.sync_copy(ids_hbm.at[pl.ds(base, TAIL)], idt_v.at[0])
        pltpu.sync_copy(h_hbm.at[pl.ds(base, TAIL)], rowt_v)
        pltpu.sync_copy(rowt_v, sums_sh.at[idt_v.at[0]], add=True)

        @pl.loop(0, TAIL, step=L)
        def _(l):
            idx = idt_v[0, pl.ds(l, L)]
            plsc.addupdate_scatter(hist_v, [idx], one16)

    plsc.subcore_barrier()

    lo = sid * SEG_PER_TILE
    pltpu.sync_copy(sums_sh.at[pl.ds(lo, SEG_PER_TILE)],
                    psum_hbm.at[cid, pl.ds(lo, SEG_PER_TILE)])
    pltpu.sync_copy(hist_v, pcnt_hbm.at[wid])


def _combine_tc(psum_ref, pcnt_ref, out_ref):
    sums = psum_ref[0] + psum_ref[1]
    cnt = jnp.maximum(jnp.sum(pcnt_ref[...], axis=0), 1.0)
    out_ref[...] = sums / cnt[:, None]


@jax.jit
def kernel(h, segment_ids):
    ids = segment_ids.astype(jnp.int32)
    psum, pcnt = _segsum_sc(h, ids)
    return pl.pallas_call(
        _combine_tc,
        out_shape=jax.ShapeDtypeStruct((S, D), jnp.float32),
    )(psum, pcnt)


# speedup vs baseline: 8.8583x; 1.0140x over previous
import dataclasses
import functools

import jax
import jax.numpy as jnp
from jax import lax
from jax.experimental import pallas as pl
from jax.experimental.pallas import tpu as pltpu
from jax.experimental.pallas import tpu_sc as plsc

N = 100000
D = 128
S = 1024
NC = 2
NS = 16
NW = NC * NS
BLK = 128
NFULL = N // BLK
TAIL = N - NFULL * BLK
KPW = NFULL // NW
REM = NFULL % NW
KMAX = KPW + 1
SEG_PER_TILE = S // NS
L = 16

_mesh = plsc.VectorSubcoreMesh(core_axis_name="c", subcore_axis_name="s")

_cp = pltpu.CompilerParams()
if "needs_layout_passes" in pltpu.CompilerParams.__dataclass_fields__:
    _cp = dataclasses.replace(_cp, needs_layout_passes=False)


@functools.partial(
    pl.kernel,
    compiler_params=_cp,
    out_type=(
        jax.ShapeDtypeStruct((NC, S, D), jnp.float32),
        jax.ShapeDtypeStruct((NW, S), jnp.float32),
    ),
    mesh=_mesh,
    scratch_types=[
        pltpu.VMEM((1, BLK), jnp.int32),
        pltpu.VMEM((1, BLK), jnp.int32),
        pltpu.VMEM((1, BLK), jnp.int32),
        pltpu.VMEM((BLK, D), jnp.float32),
        pltpu.VMEM((BLK, D), jnp.float32),
        pltpu.VMEM((BLK, D), jnp.float32),
        pltpu.VMEM((1, TAIL), jnp.int32),
        pltpu.VMEM((TAIL, D), jnp.float32),
        pltpu.VMEM((S,), jnp.float32),
        pltpu.VMEM((SEG_PER_TILE, D), jnp.float32),
        pltpu.VMEM_SHARED((S, D), jnp.float32),
        pltpu.SemaphoreType.DMA,
        pltpu.SemaphoreType.DMA,
        pltpu.SemaphoreType.DMA,
        pltpu.SemaphoreType.DMA,
        pltpu.SemaphoreType.DMA,
        pltpu.SemaphoreType.DMA,
        pltpu.SemaphoreType.DMA,
        pltpu.SemaphoreType.DMA,
        pltpu.SemaphoreType.DMA,
    ],
)
def _segsum_sc(h_hbm, ids_hbm, psum_hbm, pcnt_hbm,
               id0_v, id1_v, id2_v, rows0_v, rows1_v, rows2_v,
               idt_v, rowt_v, hist_v, zsum_v,
               sums_sh, si0, si1, si2, sr0, sr1, sr2, ss0, ss1, ss2):
    cid = lax.axis_index("c")
    sid = lax.axis_index("s")
    wid = cid * NS + sid
    start_w = wid * KPW + jnp.minimum(wid, REM)
    cnt_w = jnp.where(wid < REM, KPW + 1, KPW)

    zero16 = jnp.zeros((L,), jnp.float32)
    one16 = jnp.ones((L,), jnp.float32)

    NB = 3
    ids_v = (id0_v, id1_v, id2_v)
    rows_v = (rows0_v, rows1_v, rows2_v)
    sem_id = (si0, si1, si2)
    sem_in = (sr0, sr1, sr2)
    sem_sc = (ss0, ss1, ss2)

    def _id_desc(k):
        return pltpu.make_async_copy(
            ids_hbm.at[pl.ds((start_w + k) * BLK, BLK)], ids_v[k % NB].at[0],
            sem_id[k % NB])

    def _in_desc(k):
        return pltpu.make_async_copy(
            h_hbm.at[pl.ds((start_w + k) * BLK, BLK)], rows_v[k % NB],
            sem_in[k % NB])

    def _sc_desc(k):
        return pltpu.make_async_copy(
            rows_v[k % NB], sums_sh.at[ids_v[k % NB].at[0]], sem_sc[k % NB])

    def start_in(k):
        @pl.when(k < cnt_w)
        def _():
            _id_desc(k).start()
            _in_desc(k).start()

    def wait_sc(k):
        @pl.when(k < cnt_w)
        def _():
            _sc_desc(k).wait()

    start_in(0)
    start_in(1)

    @pl.loop(0, S, step=L)
    def _(i):
        hist_v[pl.ds(i, L)] = zero16

    @pl.loop(0, SEG_PER_TILE)
    def _(i):
        @pl.loop(0, D, step=L)
        def _(j):
            zsum_v[i, pl.ds(j, L)] = zero16

    pltpu.sync_copy(zsum_v, sums_sh.at[pl.ds(sid * SEG_PER_TILE, SEG_PER_TILE)])

    plsc.subcore_barrier()

    for k in range(KMAX):
        if k + 2 < KMAX:
            if k >= 1:
                wait_sc(k - 1)
            start_in(k + 2)

        @pl.when(k < cnt_w)
        def _(k=k):
            _id_desc(k).wait()
            _in_desc(k).wait()
            pltpu.async_copy(rows_v[k % NB], sums_sh.at[ids_v[k % NB].at[0]],
                             sem_sc[k % NB], add=True)

            @pl.loop(0, BLK, step=L)
            def _(l):
                idx = ids_v[k % NB][0, pl.ds(l, L)]
                plsc.addupdate_scatter(hist_v, [idx], one16)
    wait_sc(KMAX - 3)
    wait_sc(KMAX - 2)
    wait_sc(KMAX - 1)

    @pl.when(wid == NW - 1)
    def _():
        base = NFULL * BLK
        pltpu.sync_copy(ids_hbm.at[pl.ds(base, TAIL)], idt_v.at[0])
        pltpu.sync_copy(h_hbm.at[pl.ds(base, TAIL)], rowt_v)
        pltpu.sync_copy(rowt_v, sums_sh.at[idt_v.at[0]], add=True)

        @pl.loop(0, TAIL, step=L)
        def _(l):
            idx = idt_v[0, pl.ds(l, L)]
            plsc.addupdate_scatter(hist_v, [idx], one16)

    plsc.subcore_barrier()

    lo = sid * SEG_PER_TILE
    pltpu.sync_copy(sums_sh.at[pl.ds(lo, SEG_PER_TILE)],
                    psum_hbm.at[cid, pl.ds(lo, SEG_PER_TILE)])
    pltpu.sync_copy(hist_v, pcnt_hbm.at[wid])


def _combine_tc(psum_ref, pcnt_ref, out_ref):
    sums = psum_ref[0] + psum_ref[1]
    cnt = jnp.maximum(jnp.sum(pcnt_ref[...], axis=0), 1.0)
    out_ref[...] = sums / cnt[:, None]


@jax.jit
def kernel(h, segment_ids):
    ids = segment_ids.astype(jnp.int32)
    psum, pcnt = _segsum_sc(h, ids)
    return pl.pallas_call(
        _combine_tc,
        out_shape=jax.ShapeDtypeStruct((S, D), jnp.float32),
    )(psum, pcnt)
